# baseline (speedup 1.0000x reference)
import functools

import jax
import jax.numpy as jnp
import numpy as np
from jax import lax
from jax.experimental import pallas as pl
from jax.experimental.pallas import tpu as pltpu
from jax.experimental.pallas import tpu_sc as plsc

H = 128
W = 2048
TIME_WARP_PARA = 40
FREQ_MASK_PARA = 27
TIME_MASK_PARA = 70
FREQ_MASK_NUM = 2
TIME_MASK_NUM = 2

NUM_WORKERS = 32
LANES = 16

SPLIT = 48

ROWS_PER_WORKER = SPLIT // (NUM_WORKERS // 2)
COLS_PER_WORKER = W // 2
_CHUNK = ROWS_PER_WORKER * COLS_PER_WORKER
HALO_W = COLS_PER_WORKER + 128

_F = FREQ_MASK_PARA // 2
_T = TIME_MASK_PARA // 2
_ROW_MASKS = [((i + 1) * H // 4, (i + 1) * H // 4 + _F) for i in range(FREQ_MASK_NUM)]
_COL_MASKS = [((i + 1) * W // 4, (i + 1) * W // 4 + _T) for i in range(TIME_MASK_NUM)]


def _build_qtab():
    eps = 1e-10

    def phi(r):
        r = jnp.maximum(r, eps)
        return 0.5 * r * jnp.log(r)

    def cross_sq_dist(a, b):
        an = jnp.sum(a * a, axis=-1)[:, :, None]
        bn = jnp.sum(b * b, axis=-1)[:, None, :]
        ab = jnp.einsum('bnd,bmd->bnm', a, b)
        return an - 2.0 * ab + bn

    y = float(H // 2)
    pt = float(W // 2)
    dist = float(TIME_WARP_PARA // 2)
    src = jnp.array(
        [[[y, pt], [0.0, 0.0], [0.0, W - 1.0], [H - 1.0, 0.0], [H - 1.0, W - 1.0]]],
        dtype=jnp.float32)
    dst = src.at[0, 0, 1].add(dist)
    flows = dst - src

    c = dst
    n = 5
    matrix_a = phi(cross_sq_dist(c, c))
    ones = jnp.ones((1, n, 1), dtype=c.dtype)
    matrix_b = jnp.concatenate([c, ones], axis=2)
    left = jnp.concatenate([matrix_a, jnp.transpose(matrix_b, (0, 2, 1))], axis=1)
    nb = matrix_b.shape[2]
    right = jnp.concatenate([matrix_b, jnp.zeros((1, nb, nb), dtype=c.dtype)], axis=1)
    lhs = jnp.concatenate([left, right], axis=2)
    rhs = jnp.concatenate([flows, jnp.zeros((1, nb, 2), dtype=c.dtype)], axis=1)
    X = jnp.linalg.solve(lhs, rhs)
    w_c, v_c = X[:, :n, :], X[:, n:, :]

    yg, xg = jnp.meshgrid(jnp.linspace(0.0, H - 1.0, H),
                          jnp.linspace(0.0, W - 1.0, W), indexing='ij')
    grid = jnp.stack([yg, xg], axis=-1).reshape(H * W, 2).astype(jnp.float32)[None]
    pd = phi(cross_sq_dist(grid, c))
    rbf = jnp.einsum('bmn,bnk->bmk', pd, w_c)
    qp = jnp.concatenate([grid, jnp.ones_like(grid[..., :1])], axis=2)
    lin = jnp.einsum('bmd,bdk->bmk', qp, v_c)
    flow = (rbf + lin).reshape(H, W, 2)
    return xg.astype(jnp.float32) - flow[..., 1]


_QTAB = np.asarray(jax.jit(_build_qtab)())

_FX = np.clip(np.trunc(_QTAB).astype(np.int64), 0, W - 2)
_AX = np.clip(_QTAB - _FX.astype(np.float32), 0.0, 1.0).astype(np.float32)


def _build_sc_tables():
    fx = _FX[:SPLIT]
    ax = _AX[:SPLIT]
    r_local = (np.arange(SPLIT) % ROWS_PER_WORKER)[:, None]
    ct = (np.arange(W) // COLS_PER_WORKER) * (COLS_PER_WORKER - 128)
    lin = (r_local * HALO_W + fx - ct[None, :]).astype(np.int32)

    def chunked(t):
        return np.ascontiguousarray(
            t.reshape(SPLIT // ROWS_PER_WORKER, ROWS_PER_WORKER, 2,
                      COLS_PER_WORKER).swapaxes(1, 2)).reshape(-1)

    return chunked(lin), chunked(ax)


_SC_LIN, _SC_AX = _build_sc_tables()

TC_ROWS = H - SPLIT
_TC_D = (_FX[SPLIT:] - np.arange(W)[None, :]).astype(np.int32)
_TC_AX = _AX[SPLIT:]
_TC_SMIN = int(_TC_D.min())
_TC_SMAX = int(_TC_D.max())
_TC_NB = 8
_TC_BW = W // _TC_NB
_TC_HSW = 512
_TC_BLOCKS = []
for _b in range(_TC_NB):
    _blk = _TC_D[:, _b * _TC_BW:(_b + 1) * _TC_BW]
    _smin, _smax = int(_blk.min()), int(_blk.max())
    _start = min(max(_b * _TC_BW + _smin, 0) // 128 * 128, W - _TC_HSW)
    _TC_BLOCKS.append((_smin, _smax, _start))


def _sc_body(mel_hbm, lin_hbm, ax_hbm, out_hbm, mel_v, lin_v, ax_v, out_v, sem):
    wid = lax.axis_index('s') * 2 + lax.axis_index('c')
    i = wid // 2
    j = wid % 2
    row0 = i * ROWS_PER_WORKER
    zvec = jnp.zeros((LANES,), jnp.float32)
    lane = lax.iota(jnp.int32, LANES)

    ct = j * (COLS_PER_WORKER - 128)
    copies = [
        pltpu.async_copy(
            mel_hbm.at[0, row0 + r, pl.ds(ct, HALO_W)],
            mel_v.at[pl.ds(r * HALO_W, HALO_W)], sem)
        for r in range(ROWS_PER_WORKER)
    ]
    copies.append(
        pltpu.async_copy(lin_hbm.at[pl.ds(wid * _CHUNK, _CHUNK)], lin_v, sem))
    copies.append(
        pltpu.async_copy(ax_hbm.at[pl.ds(wid * _CHUNK, _CHUNK)], ax_v, sem))
    for cp in copies:
        cp.wait()

    for r in range(ROWS_PER_WORKER):
        @plsc.parallel_loop(0, COLS_PER_WORKER, LANES, unroll=8)
        def _(c, r=r):
            s = r * COLS_PER_WORKER + c
            lin = lin_v[pl.ds(s, LANES)]
            ax = ax_v[pl.ds(s, LANES)]
            g0 = plsc.load_gather(mel_v, [lin])
            g1 = plsc.load_gather(mel_v, [lin + 1])
            out_v[pl.ds(s, LANES)] = ax * (g1 - g0) + g0

    lo, hi = _ROW_MASKS[0]
    zs = jnp.clip(lo - row0, 0, ROWS_PER_WORKER)
    ze = jnp.clip(hi - row0, 0, ROWS_PER_WORKER)

    @plsc.parallel_loop(zs * COLS_PER_WORKER, ze * COLS_PER_WORKER, LANES)
    def _(s):
        out_v[pl.ds(s, LANES)] = zvec

    clo = jnp.where(j == 0, _COL_MASKS[0][0], _COL_MASKS[1][0] - COLS_PER_WORKER)
    for r in range(ROWS_PER_WORKER):
        rc = r * COLS_PER_WORKER + clo
        out_v[pl.ds(rc, LANES)] = zvec
        out_v[pl.ds(rc + LANES, LANES)] = zvec
        tail = rc + 2 * LANES
        cur = out_v[pl.ds(tail, LANES)]
        out_v[pl.ds(tail, LANES)] = jnp.where(lane < _T - 2 * LANES, 0.0, cur)

    outs = [
        pltpu.async_copy(
            out_v.at[pl.ds(r * COLS_PER_WORKER, COLS_PER_WORKER)],
            out_hbm.at[row0 + r, pl.ds(j * COLS_PER_WORKER, COLS_PER_WORKER)],
            sem)
        for r in range(ROWS_PER_WORKER)
    ]
    for cp in outs:
        cp.wait()


@functools.cache
def _sc_warp():
    return pl.kernel(
        _sc_body,
        mesh=plsc.VectorSubcoreMesh(core_axis_name='c', subcore_axis_name='s'),
        compiler_params=pltpu.CompilerParams(needs_layout_passes=False),
        out_type=jax.ShapeDtypeStruct((H, W), jnp.float32),
        scratch_types=[
            pltpu.VMEM((ROWS_PER_WORKER * HALO_W,), jnp.float32),
            pltpu.VMEM((_CHUNK,), jnp.int32),
            pltpu.VMEM((_CHUNK,), jnp.float32),
            pltpu.VMEM((_CHUNK,), jnp.float32),
            pltpu.SemaphoreType.DMA,
        ],
    )


def _tc_body(mel_ref, d_ref, ax_ref, out_ref):
    rows = lax.broadcasted_iota(jnp.int32, (TC_ROWS, _TC_BW), 0) + SPLIT
    cols0 = lax.broadcasted_iota(jnp.int32, (TC_ROWS, _TC_BW), 1)
    for b, (smin, smax, start) in enumerate(_TC_BLOCKS):
        hs = mel_ref[0, pl.ds(SPLIT, TC_ROWS), pl.ds(start, _TC_HSW)]
        d = d_ref[:, pl.ds(b * _TC_BW, _TC_BW)]
        ax = ax_ref[:, pl.ds(b * _TC_BW, _TC_BW)]
        off = b * _TC_BW - start
        g0 = jnp.zeros((TC_ROWS, _TC_BW), jnp.float32)
        g1 = jnp.zeros((TC_ROWS, _TC_BW), jnp.float32)
        for s in range(smin, smax + 2):
            rolled = pltpu.roll(hs, (-(s + off)) % _TC_HSW, axis=1)[:, :_TC_BW]
            if s <= smax:
                g0 = jnp.where(d == s, rolled, g0)
            if s > smin:
                g1 = jnp.where(d == (s - 1), rolled, g1)
        res = ax * (g1 - g0) + g0

        cols = cols0 + b * _TC_BW
        keep = jnp.ones((TC_ROWS, _TC_BW), jnp.bool_)
        for lo, hi in _ROW_MASKS:
            keep &= ~((rows >= lo) & (rows < hi))
        for lo, hi in _COL_MASKS:
            keep &= ~((cols >= lo) & (cols < hi))
        out_ref[:, pl.ds(b * _TC_BW, _TC_BW)] = jnp.where(keep, res, 0.0)


@functools.cache
def _tc_warp():
    return pl.pallas_call(
        _tc_body,
        grid=(1,),
        in_specs=[
            pl.BlockSpec((1, H, W), lambda i: (0, 0, 0)),
            pl.BlockSpec((TC_ROWS, W), lambda i: (0, 0)),
            pl.BlockSpec((TC_ROWS, W), lambda i: (0, 0)),
        ],
        out_specs=pl.BlockSpec((TC_ROWS, W), lambda i: (0, 0)),
        out_shape=jax.ShapeDtypeStruct((TC_ROWS, W), jnp.float32),
    )


def kernel(mel_spectrogram):
    sc_out = _sc_warp()(mel_spectrogram, jnp.asarray(_SC_LIN),
                        jnp.asarray(_SC_AX))
    tc_out = _tc_warp()(mel_spectrogram, jnp.asarray(_TC_D),
                        jnp.asarray(_TC_AX))
    out = lax.dynamic_update_slice(sc_out, tc_out, (SPLIT, 0))
    return out[None]
